# baseline (speedup 1.0000x reference)
import functools

import jax
import jax.numpy as jnp
from jax.experimental import pallas as pl
import jax.experimental.pallas.tpu as pltpu

N_W = 100000
N_GRAPHS = 512
D_IN = 128
D_OUT = 32
D_AUG = 48
BLK = 2000
N_BLK = N_W // BLK


@jax.jit
def _pool_fc(x_workload, workload_batch, fc_W, fc_b):
    batch3 = workload_batch.reshape(N_BLK, 1, BLK)
    bias2 = fc_b.reshape(1, D_OUT)
    w_aug = jnp.zeros((D_IN, D_AUG), jnp.float32).at[:, :D_OUT].set(fc_W)
    seg_iota = jax.lax.broadcasted_iota(jnp.int32, (N_GRAPHS, BLK), 0)

    def body(x_ref, b_ref, w_ref, bias_ref, iota_ref, out_ref, acc_ref):
        i = pl.program_id(0)

        @pl.when(i == 0)
        def _init():
            acc_ref[...] = jnp.zeros_like(acc_ref)

        x = jnp.maximum(x_ref[...], 0.0)
        y = jax.lax.dot_general(
            x, w_ref[...], (((1,), (0,)), ((), ())),
            preferred_element_type=jnp.float32)
        lane = jax.lax.broadcasted_iota(jnp.int32, (1, D_AUG), 1)
        y = y + jnp.where(lane == D_OUT, 1.0, 0.0)
        seg = b_ref[0]
        onehot = (iota_ref[...] == seg).astype(jnp.float32)
        acc_ref[...] += jax.lax.dot_general(
            onehot, y, (((1,), (0,)), ((), ())),
            preferred_element_type=jnp.float32)

        @pl.when(i == N_BLK - 1)
        def _finish():
            c = jnp.maximum(acc_ref[:, D_OUT:D_OUT + 1], 1.0)
            out_ref[...] = acc_ref[:, :D_OUT] / c + bias_ref[...]

    return pl.pallas_call(
        body,
        grid=(N_BLK,),
        in_specs=[
            pl.BlockSpec((BLK, D_IN), lambda i: (i, 0)),
            pl.BlockSpec((1, 1, BLK), lambda i: (i, 0, 0)),
            pl.BlockSpec((D_IN, D_AUG), lambda i: (0, 0)),
            pl.BlockSpec((1, D_OUT), lambda i: (0, 0)),
            pl.BlockSpec((N_GRAPHS, BLK), lambda i: (0, 0)),
        ],
        out_specs=pl.BlockSpec((N_GRAPHS, D_OUT), lambda i: (0, 0)),
        out_shape=jax.ShapeDtypeStruct((N_GRAPHS, D_OUT), jnp.float32),
        scratch_shapes=[
            pltpu.VMEM((N_GRAPHS, D_AUG), jnp.float32),
        ],
    )(x_workload, batch3, w_aug, bias2, seg_iota)


def kernel(x_workload, x_vm, x_host, edge_index_assigned, edge_index_runs,
           workload_batch, conv1_gcn_W, conv1_gcn_b, conv1_sage_Wl,
           conv1_sage_Wr, conv1_sage_b, conv2_gcn_W, conv2_gcn_b,
           conv2_sage_Wl, conv2_sage_Wr, conv2_sage_b, fc_W, fc_b):
    return _pool_fc(x_workload, workload_batch, fc_W, fc_b)
